# baseline (speedup 1.0000x reference)
import functools

import jax
import jax.numpy as jnp
import numpy as np
from jax import lax
from jax.experimental import pallas as pl
from jax.experimental.pallas import tpu as pltpu
from jax.experimental.pallas import tpu_sc as plsc

N, E, FIN, FOUT = 10000, 320000, 128, 256
NC, NS, L = 2, 16, 16
NW = NC * NS
CW = 80
RPW = E // (NW * CW)
BLK = 25
NP = 10240
SEG = NP // NS
FP = FIN // 2

_mesh = plsc.VectorSubcoreMesh(
    core_axis_name="c", subcore_axis_name="s", num_cores=NC, num_subcores=NS)
_sc_params = pltpu.CompilerParams(use_tc_tiling_on_sc=False)

_PERM = np.concatenate(
    [np.concatenate([16 * r + np.arange(16), 64 + 16 * r + np.arange(16)])
     for r in range(4)]).astype(np.int32)


def _bcast_lane(v16, l):
    return lax.gather(
        v16, jnp.full((L, 1), l, jnp.int32),
        lax.GatherDimensionNumbers(
            offset_dims=(), collapsed_slice_dims=(0,), start_index_map=(0,)),
        (1,), mode=lax.GatherScatterMode.PROMISE_IN_BOUNDS)


@functools.partial(
    pl.kernel,
    out_type=jax.ShapeDtypeStruct((NC, NP), jnp.float32),
    mesh=_mesh,
    scratch_types=[
        pltpu.VMEM((RPW, CW), jnp.int32),
        pltpu.VMEM((RPW, CW), jnp.float32),
        pltpu.VMEM_SHARED((NP,), jnp.float32),
    ],
    compiler_params=_sc_params,
)
def _deg_kernel(dst2, ew2, zcol, out, dbuf, wbuf, deg_sh):
    c = lax.axis_index("c")
    s = lax.axis_index("s")
    w = s * NC + c
    pltpu.sync_copy(zcol, deg_sh.at[pl.ds(s * SEG, SEG)])
    plsc.subcore_barrier()
    base = w * RPW
    pltpu.sync_copy(dst2.at[pl.ds(base, RPW)], dbuf)
    pltpu.sync_copy(ew2.at[pl.ds(base, RPW)], wbuf)

    def body(j, carry):
        pltpu.sync_copy(wbuf.at[j], deg_sh.at[dbuf.at[j]], add=True)
        return carry

    lax.fori_loop(0, RPW, body, 0)
    plsc.subcore_barrier()
    pltpu.sync_copy(deg_sh.at[pl.ds(s * SEG, SEG)], out.at[c, pl.ds(s * SEG, SEG)])


@functools.partial(
    pl.kernel,
    out_type=jax.ShapeDtypeStruct((NC, NP, FIN), jnp.float32),
    mesh=_mesh,
    scratch_types=[
        pltpu.VMEM((BLK, CW), jnp.int32),
        pltpu.VMEM((BLK, CW), jnp.int32),
        pltpu.VMEM((BLK, CW), jnp.float32),
        pltpu.VMEM((BLK, CW), jnp.float32),
        pltpu.VMEM((CW, FP), jnp.float32),
        pltpu.VMEM((CW, FP), jnp.float32),
        pltpu.VMEM((CW, FIN), jnp.float32),
        pltpu.VMEM((CW, FIN), jnp.float32),
        pltpu.VMEM((SEG,), jnp.float32),
        pltpu.VMEM((SEG,), jnp.float32),
        pltpu.VMEM_SHARED((NP, FIN), jnp.float32),
        pltpu.VMEM_SHARED((NP,), jnp.float32),
        pltpu.SemaphoreType.DMA,
        pltpu.SemaphoreType.DMA,
    ],
    compiler_params=_sc_params,
)
def _agg_kernel(src2, dst2, ew2, xq, degp, zrows, out,
                sbuf, dbuf, wbuf, dvb, gbuf0, gbuf1, rows0, rows1,
                b0, b1, acc_sh, dinv_sh, sem0, sem1):
    c = lax.axis_index("c")
    s = lax.axis_index("s")
    w = s * NC + c
    pltpu.sync_copy(zrows, acc_sh.at[pl.ds(s * SEG, SEG)])
    pltpu.sync_copy(degp.at[0, pl.ds(s * SEG, SEG)], b0)
    pltpu.sync_copy(degp.at[1, pl.ds(s * SEG, SEG)], b1)
    for k in range(SEG // L):
        sl = pl.ds(k * L, L)
        d = b0[sl] + b1[sl] + 1.0
        i = lax.bitcast_convert_type(d, jnp.int32)
        i = jnp.int32(0x5F3759DF) - lax.shift_right_logical(i, jnp.int32(1))
        y = lax.bitcast_convert_type(i, jnp.float32)
        hd = 0.5 * d
        y = y * (1.5 - hd * y * y)
        y = y * (1.5 - hd * y * y)
        b0[sl] = y
    pltpu.sync_copy(b0, dinv_sh.at[pl.ds(s * SEG, SEG)])
    plsc.subcore_barrier()
    base = w * RPW

    def start(j, gbuf, sem):
        pltpu.async_copy(xq.at[sbuf.at[j]], gbuf, sem)

    def drain(j, gbuf, rows, sem):
        pltpu.sync_copy(dinv_sh.at[sbuf.at[j]], dvb.at[j])
        pltpu.make_async_copy(xq.at[sbuf.at[j]], gbuf, sem).wait()
        for g in range(CW // L):
            gs = pl.ds(g * L, L)
            w16 = wbuf[j, gs] * dvb[j, gs]
            for l in range(L):
                bc = _bcast_lane(w16, l)
                e = g * L + l
                for r in range(FP // L):
                    u = lax.bitcast_convert_type(gbuf[e, pl.ds(r * L, L)],
                                                 jnp.uint32)
                    lo = lax.bitcast_convert_type(u << jnp.uint32(16),
                                                  jnp.float32)
                    hi = lax.bitcast_convert_type(u, jnp.float32)
                    rows[e, pl.ds(2 * r * L, L)] = lo * bc
                    rows[e, pl.ds((2 * r + 1) * L, L)] = hi * bc
        pltpu.sync_copy(rows, acc_sh.at[dbuf.at[j]], add=True)

    def block(b, carry):
        r0 = base + b * BLK
        pltpu.sync_copy(src2.at[pl.ds(r0, BLK)], sbuf)
        pltpu.sync_copy(dst2.at[pl.ds(r0, BLK)], dbuf)
        pltpu.sync_copy(ew2.at[pl.ds(r0, BLK)], wbuf)
        start(0, gbuf0, sem0)
        start(1, gbuf1, sem1)

        def body(i, carry2):
            j0 = 2 * i
            drain(j0, gbuf0, rows0, sem0)

            @pl.when(j0 + 2 < BLK)
            def _():
                start(j0 + 2, gbuf0, sem0)

            @pl.when(j0 + 1 < BLK)
            def _():
                drain(j0 + 1, gbuf1, rows1, sem1)

                @pl.when(j0 + 3 < BLK)
                def _():
                    start(j0 + 3, gbuf1, sem1)

            return carry2

        lax.fori_loop(0, (BLK + 1) // 2, body, 0)
        return carry

    lax.fori_loop(0, RPW // BLK, block, 0)
    plsc.subcore_barrier()
    pltpu.sync_copy(acc_sh.at[pl.ds(s * SEG, SEG)],
                    out.at[c, pl.ds(s * SEG, SEG)])


def _dense_body(dp, a0, a1, x, h, m, nzr, nh, wt, cb, wb, y_ref, hn_ref):
    bf = jnp.bfloat16
    ct = (((1,), (1,)), ((), ()))
    d = dp[:, 0:1] + dp[:, 1:2] + 1.0
    dvv = lax.rsqrt(d)
    hh = h[...]
    pre = dvv * (a0[...][0] + a1[...][0]) + (dvv * dvv) * x[...]
    gg = jnp.dot(pre.astype(bf), m[...],
                 preferred_element_type=jnp.float32) + cb[...]
    hzr = lax.dot_general(hh.astype(bf), nzr[...], ct,
                          preferred_element_type=jnp.float32)
    z = jax.nn.sigmoid(gg[:, :FOUT] + hzr[:, :FOUT])
    r = jax.nn.sigmoid(gg[:, FOUT:2 * FOUT] + hzr[:, FOUT:])
    ht = jnp.tanh(gg[:, 2 * FOUT:] +
                  lax.dot_general((hh * r).astype(bf), nh[...], ct,
                                  preferred_element_type=jnp.float32))
    hn = z * hh + (1.0 - z) * ht
    hn_ref[...] = hn
    y_ref[...] = (lax.dot_general(jax.nn.relu(hn).astype(bf), wt[...], ct,
                                  preferred_element_type=jnp.float32) + wb[...])


_BN = 400
_GRID = N // _BN


def _row_spec(cols):
    return pl.BlockSpec((_BN, cols), lambda i: (i, 0))


def _full_spec(r, cols):
    return pl.BlockSpec((r, cols), lambda i: (0, 0))


_dense_call = pl.pallas_call(
    _dense_body,
    grid=(_GRID,),
    in_specs=[_row_spec(2),
              pl.BlockSpec((1, _BN, FIN), lambda i: (0, i, 0)),
              pl.BlockSpec((1, _BN, FIN), lambda i: (1, i, 0)),
              _row_spec(FIN), _row_spec(FOUT),
              _full_spec(FIN, 3 * FOUT), _full_spec(2 * FOUT, FOUT),
              _full_spec(FOUT, FOUT), _full_spec(FIN, FOUT),
              _full_spec(1, 3 * FOUT), _full_spec(1, FIN)],
    out_specs=[_row_spec(FIN), _row_spec(FOUT)],
    out_shape=[jax.ShapeDtypeStruct((N, FIN), jnp.float32),
               jax.ShapeDtypeStruct((N, FOUT), jnp.float32)],
)


def kernel(g, node_feat, edge_weight, hidden_state, Wz, bz, Wr, br, Wh, bh,
           LzW, Lzb, LrW, Lrb, LhW, Lhb, WlinW, Wlinb):
    src2 = lax.optimization_barrier(g)[0].reshape(E // CW, CW)
    dst2 = g[1].reshape(E // CW, CW)
    ew2 = edge_weight.reshape(E // CW, CW)
    zcol = jnp.zeros((SEG,), jnp.float32)
    zrows = jnp.zeros((SEG, FIN), jnp.float32)

    u = lax.bitcast_convert_type(node_feat, jnp.uint32)
    t = (u + jnp.uint32(0x7FFF) + ((u >> jnp.uint32(16)) & jnp.uint32(1))) \
        & jnp.uint32(0xFFFF0000)
    xq = lax.bitcast_convert_type(
        (t[:, :FP] >> jnp.uint32(16)) | t[:, FP:], jnp.float32)
    perm = jnp.asarray(_PERM)
    xP = node_feat[:, perm]

    deg_part = _deg_kernel(dst2, ew2, zcol)
    agg_part = _agg_kernel(src2, dst2, ew2, xq, deg_part, zrows)

    ct = (((1,), (1,)), ((), ()))

    def fold(W, b, LW, Lb):
        LA = LW[:, :FOUT]
        cc = lax.dot_general(b.reshape(1, FOUT), LA, ct) + Lb.reshape(1, FOUT)
        return (lax.dot_general(W, LA, ct), cc, LW[:, FOUT:])

    Mz, cz, LBz = fold(Wz, bz, LzW, Lzb)
    Mr, cr, LBr = fold(Wr, br, LrW, Lrb)
    Mh, ch, LBh = fold(Wh, bh, LhW, Lhb)
    M = jnp.concatenate([Mz, Mr, Mh], axis=1)
    c = jnp.concatenate([cz, cr, ch], axis=1)
    LBzr = jnp.concatenate([LBz, LBr], axis=0)
    MP = M[perm, :]

    bf = jnp.bfloat16
    degT = deg_part[:, :N].T
    y, hn = _dense_call(degT,
                        agg_part, agg_part, xP, hidden_state,
                        MP.astype(bf), LBzr.astype(bf), LBh.astype(bf),
                        WlinW.astype(bf), c, Wlinb.reshape(1, FIN))
    return (y, hn)
